# baseline (speedup 1.0000x reference)
import functools

import jax
import jax.numpy as jnp
from jax import lax
from jax.experimental import layout as jlayout
from jax.experimental import pallas as pl
from jax.experimental.pallas import tpu as pltpu
from jax.experimental.pallas import tpu_sc as plsc

B, V, C = 8, 50000, 16
H = W = 512
HW = H * W

NC, NS = 2, 16
NW = NC * NS
LANES = 16

PIX_PER_W = HW // NW
PIX_SUB = 4096
TBLK = 2000
TPITCH = TBLK + 1
N_TTASK = B * (V // TBLK)
TTASK_PER_W = (N_TTASK + NW - 1) // NW

ROWS = B * C
ROW_PER_W = ROWS // NW
PCH = 4096
PROWS = PCH // W
N_PCH = HW // PCH

_mesh = plsc.VectorSubcoreMesh(core_axis_name="c", subcore_axis_name="s")


def _wid():
    return lax.axis_index("s") * NC + lax.axis_index("c")


def _prep_body(vv_hbm, ind_hbm, v2i_hbm, cidx_hbm, tblt_hbm,
               ind_v, v2i_v, cidx_v, tin_v, tcol_v):
    w = _wid()

    pltpu.sync_copy(ind_hbm, ind_v)
    for sub in range(PIX_PER_W // PIX_SUB):
        poff = w * PIX_PER_W + sub * PIX_SUB
        row0 = poff // W
        pltpu.sync_copy(v2i_hbm.at[pl.ds(row0, PIX_SUB // W), :], v2i_v)

        @plsc.parallel_loop(0, PIX_SUB // LANES, unroll=8)
        def _(j):
            idx = v2i_v[j // (W // LANES), pl.ds((j % (W // LANES)) * LANES,
                                                 LANES)]
            cidx_v[pl.ds(j * LANES, LANES)] = plsc.load_gather(ind_v, [idx])
        pltpu.sync_copy(cidx_v, cidx_hbm.at[pl.ds(poff, PIX_SUB)])

    iota = lax.iota(jnp.int32, LANES)

    def ttask(t):
        b = t // (V // TBLK)
        roff = (t % (V // TBLK)) * TBLK
        pltpu.sync_copy(vv_hbm.at[b, pl.ds(roff, TBLK), :], tin_v)

        @plsc.parallel_loop(0, TBLK, unroll=8)
        def _(p):
            val = tin_v[p, :]
            plsc.store_scatter(
                tcol_v, [iota, jnp.full((LANES,), p, jnp.int32)], val)
        pltpu.sync_copy(tcol_v.at[:, pl.ds(0, TBLK)],
                        tblt_hbm.at[pl.ds(b * C, C), pl.ds(roff, TBLK)])

    def touter(k, _):
        t = w + k * NW

        @pl.when(t < N_TTASK)
        def _():
            ttask(t)

        return 0

    lax.fori_loop(0, TTASK_PER_W, touter, 0)


def _gather_body(tblt_hbm, cidx_hbm, out_hbm,
                 ta_v, tb_v, c0_v, c1_v, o00_v, o10_v, o01_v, o11_v,
                 si0, si1, so0, so1):
    w = _wid()
    cbufs = ((c0_v, si0), (c1_v, si1))
    obufs = ((o00_v, o10_v, so0), (o01_v, o11_v, so1))

    for half in range(ROW_PER_W // 2):
        r0 = w * ROW_PER_W + half * 2
        b0 = r0 // C
        c0 = r0 % C
        pltpu.sync_copy(tblt_hbm.at[r0], ta_v)
        pltpu.sync_copy(tblt_hbm.at[r0 + 1], tb_v)
        pltpu.async_copy(cidx_hbm.at[pl.ds(0, PCH)], c0_v, si0)

        def sub(kk, i):
            cin, si = cbufs[i]
            oa, ob, so = obufs[i]
            cnx, snx = cbufs[1 - i]
            ch = kk * 2 + i
            pltpu.make_async_copy(cidx_hbm.at[pl.ds(0, PCH)], cin, si).wait()

            @pl.when(ch + 1 < N_PCH)
            def _():
                pltpu.async_copy(
                    cidx_hbm.at[pl.ds((ch + 1) * PCH, PCH)], cnx, snx)

            @pl.when(kk > 0)
            def _():
                pltpu.make_async_copy(
                    oa, out_hbm.at[b0, c0, pl.ds(0, PROWS), :], so).wait()
                pltpu.make_async_copy(
                    ob, out_hbm.at[b0, c0 + 1, pl.ds(0, PROWS), :], so).wait()

            @plsc.parallel_loop(0, PCH // LANES, unroll=8)
            def _(j):
                r = j // (W // LANES)
                s = pl.ds((j % (W // LANES)) * LANES, LANES)
                idx = cin[pl.ds(j * LANES, LANES)]
                oa[r, s] = plsc.load_gather(ta_v, [idx])
                ob[r, s] = plsc.load_gather(tb_v, [idx])

            prow = ch * PROWS
            pltpu.async_copy(oa, out_hbm.at[b0, c0, pl.ds(prow, PROWS), :], so)
            pltpu.async_copy(
                ob, out_hbm.at[b0, c0 + 1, pl.ds(prow, PROWS), :], so)

        def kk_body(kk, _):
            sub(kk, 0)
            sub(kk, 1)
            return 0

        lax.fori_loop(0, N_PCH // 2, kk_body, 0)
        for oa, ob, so in obufs:
            pltpu.make_async_copy(
                oa, out_hbm.at[b0, c0, pl.ds(0, PROWS), :], so).wait()
            pltpu.make_async_copy(
                ob, out_hbm.at[b0, c0 + 1, pl.ds(0, PROWS), :], so).wait()


_params = pltpu.CompilerParams(use_tc_tiling_on_sc=False,
                               needs_layout_passes=False)

_prep = functools.partial(
    pl.kernel,
    out_type=(
        jax.ShapeDtypeStruct((HW,), jnp.int32),
        jax.ShapeDtypeStruct((B * C, V), jnp.float32),
    ),
    mesh=_mesh,
    compiler_params=_params,
    scratch_types=[
        pltpu.VMEM((V,), jnp.int32),
        pltpu.VMEM((PIX_SUB // W, W), jnp.int32),
        pltpu.VMEM((PIX_SUB,), jnp.int32),
        pltpu.VMEM((TBLK, C), jnp.float32),
        pltpu.VMEM((C, TPITCH), jnp.float32),
    ],
)(_prep_body)

_gather = functools.partial(
    pl.kernel,
    out_type=jax.ShapeDtypeStruct((B, C, H, W), jnp.float32),
    mesh=_mesh,
    compiler_params=_params,
    scratch_types=[
        pltpu.VMEM((V,), jnp.float32),
        pltpu.VMEM((V,), jnp.float32),
        pltpu.VMEM((PCH,), jnp.int32),
        pltpu.VMEM((PCH,), jnp.int32),
        pltpu.VMEM((PROWS, W), jnp.float32),
        pltpu.VMEM((PROWS, W), jnp.float32),
        pltpu.VMEM((PROWS, W), jnp.float32),
        pltpu.VMEM((PROWS, W), jnp.float32),
        pltpu.SemaphoreType.DMA,
        pltpu.SemaphoreType.DMA,
        pltpu.SemaphoreType.DMA,
        pltpu.SemaphoreType.DMA,
    ],
)(_gather_body)


def _impl(vertex_values, indices, v2i_idx):
    ind32 = indices if indices.dtype == jnp.int32 else indices.astype(jnp.int32)
    v2i = v2i_idx if v2i_idx.dtype == jnp.int32 else v2i_idx.astype(jnp.int32)
    cidx, tblt = _prep(vertex_values, ind32, v2i)
    return _gather(tblt, cidx)


@functools.cache
def _jitted():
    fmt = jlayout.Format(
        jlayout.Layout(major_to_minor=(0, 1, 2, 3), tiling=((8,),)),
        jax.sharding.SingleDeviceSharding(jax.devices()[0]))
    return jax.jit(_impl, out_shardings=fmt)


def kernel(vertex_values, indices, v2i_idx):
    return _jitted()(vertex_values, indices, v2i_idx)


# speedup vs baseline: 1.0155x; 1.0155x over previous
import functools

import jax
import jax.numpy as jnp
from jax import lax
from jax.experimental import pallas as pl
from jax.experimental.pallas import tpu as pltpu
from jax.experimental.pallas import tpu_sc as plsc

B, V, C = 8, 50000, 16
H = W = 512
HW = H * W

NC, NS = 2, 16
NW = NC * NS
LANES = 16

PIX_PER_W = HW // NW
PIX_SUB = 4096
TBLK = 2000
TPITCH = TBLK + 1
N_TTASK = B * (V // TBLK)
TTASK_PER_W = (N_TTASK + NW - 1) // NW

ROWS = B * C
ROW_PER_W = ROWS // NW
PCH = 4096
PROWS = PCH // W
N_PCH = HW // PCH

_mesh = plsc.VectorSubcoreMesh(core_axis_name="c", subcore_axis_name="s")


def _wid():
    return lax.axis_index("s") * NC + lax.axis_index("c")


def _cidx_body(ind_hbm, v2i_hbm, cidx_hbm, ind_v, v2i_v, cidx_v):
    w = _wid()

    pltpu.sync_copy(ind_hbm, ind_v)
    for sub in range(PIX_PER_W // PIX_SUB):
        poff = w * PIX_PER_W + sub * PIX_SUB
        row0 = pl.multiple_of(poff // W, 8)
        pltpu.sync_copy(v2i_hbm.at[pl.ds(row0, PIX_SUB // W), :], v2i_v)

        @plsc.parallel_loop(0, PIX_SUB // LANES, unroll=8)
        def _(j):
            idx = v2i_v[j // (W // LANES), pl.ds((j % (W // LANES)) * LANES,
                                                 LANES)]
            cidx_v[pl.ds(j * LANES, LANES)] = plsc.load_gather(ind_v, [idx])
        pltpu.sync_copy(cidx_v, cidx_hbm.at[pl.ds(poff, PIX_SUB)])


def _prep_body(vv_hbm, tblt_hbm, tin_v, tcol_v):
    w = _wid()

    iota = lax.iota(jnp.int32, LANES)

    def ttask(t):
        b = t // (V // TBLK)
        roff = (t % (V // TBLK)) * TBLK
        pltpu.sync_copy(vv_hbm.at[b, pl.ds(roff, TBLK), :], tin_v)

        @plsc.parallel_loop(0, TBLK, unroll=8)
        def _(p):
            val = tin_v[p, :]
            plsc.store_scatter(
                tcol_v, [iota, jnp.full((LANES,), p, jnp.int32)], val)
        pltpu.sync_copy(tcol_v.at[:, pl.ds(0, TBLK)],
                        tblt_hbm.at[pl.ds(b * C, C), pl.ds(roff, TBLK)])

    def touter(k, _):
        t = w + k * NW

        @pl.when(t < N_TTASK)
        def _():
            ttask(t)

        return 0

    lax.fori_loop(0, TTASK_PER_W, touter, 0)


def _gather_body(tblt_hbm, cidx_hbm, out_hbm,
                 ta_v, tb_v, c0_v, c1_v, o00_v, o10_v, o01_v, o11_v,
                 si0, si1, so0, so1):
    w = _wid()
    cbufs = ((c0_v, si0), (c1_v, si1))
    obufs = ((o00_v, o10_v, so0), (o01_v, o11_v, so1))

    for half in range(ROW_PER_W // 2):
        r0 = w * ROW_PER_W + half * 2
        b0 = r0 // C
        c0 = r0 % C
        pltpu.sync_copy(tblt_hbm.at[r0], ta_v)
        pltpu.sync_copy(tblt_hbm.at[r0 + 1], tb_v)
        pltpu.async_copy(cidx_hbm.at[pl.ds(0, PCH)], c0_v, si0)

        def sub(kk, i):
            cin, si = cbufs[i]
            oa, ob, so = obufs[i]
            cnx, snx = cbufs[1 - i]
            ch = kk * 2 + i
            pltpu.make_async_copy(cidx_hbm.at[pl.ds(0, PCH)], cin, si).wait()

            @pl.when(ch + 1 < N_PCH)
            def _():
                pltpu.async_copy(
                    cidx_hbm.at[pl.ds((ch + 1) * PCH, PCH)], cnx, snx)

            @pl.when(kk > 0)
            def _():
                pltpu.make_async_copy(
                    oa, out_hbm.at[b0, c0, pl.ds(0, PROWS), :], so).wait()
                pltpu.make_async_copy(
                    ob, out_hbm.at[b0, c0 + 1, pl.ds(0, PROWS), :], so).wait()

            @plsc.parallel_loop(0, PCH // LANES, unroll=8)
            def _(j):
                r = j // (W // LANES)
                s = pl.ds((j % (W // LANES)) * LANES, LANES)
                idx = cin[pl.ds(j * LANES, LANES)]
                oa[r, s] = plsc.load_gather(ta_v, [idx])
                ob[r, s] = plsc.load_gather(tb_v, [idx])

            prow = ch * PROWS
            pltpu.async_copy(oa, out_hbm.at[b0, c0, pl.ds(prow, PROWS), :], so)
            pltpu.async_copy(
                ob, out_hbm.at[b0, c0 + 1, pl.ds(prow, PROWS), :], so)

        def kk_body(kk, _):
            sub(kk, 0)
            sub(kk, 1)
            return 0

        lax.fori_loop(0, N_PCH // 2, kk_body, 0)
        for oa, ob, so in obufs:
            pltpu.make_async_copy(
                oa, out_hbm.at[b0, c0, pl.ds(0, PROWS), :], so).wait()
            pltpu.make_async_copy(
                ob, out_hbm.at[b0, c0 + 1, pl.ds(0, PROWS), :], so).wait()


_params = pltpu.CompilerParams(use_tc_tiling_on_sc=False,
                               needs_layout_passes=False)
_params_tiled = pltpu.CompilerParams(use_tc_tiling_on_sc=True,
                                     needs_layout_passes=False)

_cidx = functools.partial(
    pl.kernel,
    out_type=jax.ShapeDtypeStruct((HW,), jnp.int32),
    mesh=_mesh,
    compiler_params=_params_tiled,
    scratch_types=[
        pltpu.VMEM((V,), jnp.int32),
        pltpu.VMEM((PIX_SUB // W, W), jnp.int32),
        pltpu.VMEM((PIX_SUB,), jnp.int32),
    ],
)(_cidx_body)

_prep = functools.partial(
    pl.kernel,
    out_type=jax.ShapeDtypeStruct((B * C, V), jnp.float32),
    mesh=_mesh,
    compiler_params=_params,
    scratch_types=[
        pltpu.VMEM((TBLK, C), jnp.float32),
        pltpu.VMEM((C, TPITCH), jnp.float32),
    ],
)(_prep_body)

_gather = functools.partial(
    pl.kernel,
    out_type=jax.ShapeDtypeStruct((B, C, H, W), jnp.float32),
    mesh=_mesh,
    compiler_params=_params,
    scratch_types=[
        pltpu.VMEM((V,), jnp.float32),
        pltpu.VMEM((V,), jnp.float32),
        pltpu.VMEM((PCH,), jnp.int32),
        pltpu.VMEM((PCH,), jnp.int32),
        pltpu.VMEM((PROWS, W), jnp.float32),
        pltpu.VMEM((PROWS, W), jnp.float32),
        pltpu.VMEM((PROWS, W), jnp.float32),
        pltpu.VMEM((PROWS, W), jnp.float32),
        pltpu.SemaphoreType.DMA,
        pltpu.SemaphoreType.DMA,
        pltpu.SemaphoreType.DMA,
        pltpu.SemaphoreType.DMA,
    ],
)(_gather_body)


@jax.jit
def kernel(vertex_values, indices, v2i_idx):
    ind32 = indices if indices.dtype == jnp.int32 else indices.astype(jnp.int32)
    v2i = v2i_idx if v2i_idx.dtype == jnp.int32 else v2i_idx.astype(jnp.int32)
    cidx = _cidx(ind32, v2i)
    tblt = _prep(vertex_values)
    return _gather(tblt, cidx)


# speedup vs baseline: 1.4018x; 1.3804x over previous
import functools

import jax
import jax.numpy as jnp
from jax import lax
from jax.experimental import pallas as pl
from jax.experimental.pallas import tpu as pltpu
from jax.experimental.pallas import tpu_sc as plsc

B, V, C = 8, 50000, 16
H = W = 512
HW = H * W

NC, NS = 2, 16
NW = NC * NS
LANES = 16

PIX_PER_W = HW // NW
PIX_SUB = 4096

VCH = 3328
N_VCH = 15
VT0 = N_VCH * VCH
VTAIL = V - VT0
N_RTASK = B * 2 * N_VCH
RTASK_PER_W = (N_RTASK + NW - 1) // NW

ROWS = B * C
ROW_PER_W = ROWS // NW
PCH = 4096
PROWS = PCH // W
N_PCH = HW // PCH

_mesh = plsc.VectorSubcoreMesh(core_axis_name="c", subcore_axis_name="s")


def _wid():
    return lax.axis_index("s") * NC + lax.axis_index("c")


def _cidx_body(ind_hbm, v2i_hbm, cidx_hbm, ind_v, v2i_v, cidx_v):
    w = _wid()

    pltpu.sync_copy(ind_hbm, ind_v)
    for sub in range(PIX_PER_W // PIX_SUB):
        poff = w * PIX_PER_W + sub * PIX_SUB
        row0 = pl.multiple_of(poff // W, 8)
        pltpu.sync_copy(v2i_hbm.at[pl.ds(row0, PIX_SUB // W), :], v2i_v)

        @plsc.parallel_loop(0, PIX_SUB // LANES, unroll=8)
        def _(j):
            idx = v2i_v[j // (W // LANES), pl.ds((j % (W // LANES)) * LANES,
                                                 LANES)]
            cidx_v[pl.ds(j * LANES, LANES)] = plsc.load_gather(ind_v, [idx])
        pltpu.sync_copy(cidx_v, cidx_hbm.at[pl.ds(poff, PIX_SUB)])


def _relay_body(vvt_hbm, tail_hbm, tblt_hbm, slab_v, sem):
    w = _wid()

    def task(b, cb, v0, ln):
        sl = slab_v.at[:, pl.ds(0, ln)]
        pltpu.sync_copy(
            vvt_hbm.at[b, pl.ds(pl.multiple_of(cb * 8, 8), 8),
                       pl.ds(v0, ln)], sl)
        for cr in range(8):
            off = (b * C + cb * 8 + cr) * V + v0
            pltpu.async_copy(slab_v.at[cr, pl.ds(0, ln)],
                             tblt_hbm.at[pl.ds(pl.multiple_of(off, 8), ln)],
                             sem)
        for _ in range(8):
            pltpu.make_async_copy(slab_v.at[0, pl.ds(0, ln)],
                                  tblt_hbm.at[pl.ds(0, ln)], sem).wait()

    def router(k, _):
        t = w + k * NW

        @pl.when(t < N_RTASK)
        def _():
            b = t // (2 * N_VCH)
            rem = t % (2 * N_VCH)
            task(b, rem // N_VCH,
                 pl.multiple_of((rem % N_VCH) * VCH, 128), VCH)

        return 0

    lax.fori_loop(0, RTASK_PER_W, router, 0)

    @pl.when(w < B * 2)
    def _():
        b = w // 2
        cb = w % 2
        sl = slab_v.at[:, pl.ds(0, VTAIL)]
        pltpu.sync_copy(
            tail_hbm.at[b, pl.ds(pl.multiple_of(cb * 8, 8), 8), :], sl)
        for cr in range(8):
            off = (b * C + cb * 8 + cr) * V + VT0
            pltpu.async_copy(slab_v.at[cr, pl.ds(0, VTAIL)],
                             tblt_hbm.at[pl.ds(pl.multiple_of(off, 8),
                                               VTAIL)], sem)
        for _ in range(8):
            pltpu.make_async_copy(slab_v.at[0, pl.ds(0, VTAIL)],
                                  tblt_hbm.at[pl.ds(0, VTAIL)], sem).wait()


def _gather_body(tblt_hbm, cidx_hbm, out_hbm,
                 ta_v, tb_v, c0_v, c1_v, o00_v, o10_v, o01_v, o11_v,
                 si0, si1, so0, so1):
    w = _wid()
    cbufs = ((c0_v, si0), (c1_v, si1))
    obufs = ((o00_v, o10_v, so0), (o01_v, o11_v, so1))

    for half in range(ROW_PER_W // 2):
        r0 = w * ROW_PER_W + half * 2
        b0 = r0 // C
        c0 = r0 % C
        pltpu.sync_copy(tblt_hbm.at[pl.ds(pl.multiple_of(r0 * V, 8), V)], ta_v)
        pltpu.sync_copy(
            tblt_hbm.at[pl.ds(pl.multiple_of((r0 + 1) * V, 8), V)], tb_v)
        pltpu.async_copy(cidx_hbm.at[pl.ds(0, PCH)], c0_v, si0)

        def sub(kk, i):
            cin, si = cbufs[i]
            oa, ob, so = obufs[i]
            cnx, snx = cbufs[1 - i]
            ch = kk * 2 + i
            pltpu.make_async_copy(cidx_hbm.at[pl.ds(0, PCH)], cin, si).wait()

            @pl.when(ch + 1 < N_PCH)
            def _():
                pltpu.async_copy(
                    cidx_hbm.at[pl.ds((ch + 1) * PCH, PCH)], cnx, snx)

            @pl.when(kk > 0)
            def _():
                pltpu.make_async_copy(
                    oa, out_hbm.at[b0, c0, pl.ds(0, PROWS), :], so).wait()
                pltpu.make_async_copy(
                    ob, out_hbm.at[b0, c0 + 1, pl.ds(0, PROWS), :], so).wait()

            @plsc.parallel_loop(0, PCH // LANES, unroll=8)
            def _(j):
                r = j // (W // LANES)
                s = pl.ds((j % (W // LANES)) * LANES, LANES)
                idx = cin[pl.ds(j * LANES, LANES)]
                oa[r, s] = plsc.load_gather(ta_v, [idx])
                ob[r, s] = plsc.load_gather(tb_v, [idx])

            prow = pl.multiple_of(ch * PROWS, 8)
            pltpu.async_copy(oa, out_hbm.at[b0, c0, pl.ds(prow, PROWS), :], so)
            pltpu.async_copy(
                ob, out_hbm.at[b0, c0 + 1, pl.ds(prow, PROWS), :], so)

        def kk_body(kk, _):
            sub(kk, 0)
            sub(kk, 1)
            return 0

        lax.fori_loop(0, N_PCH // 2, kk_body, 0)
        for oa, ob, so in obufs:
            pltpu.make_async_copy(
                oa, out_hbm.at[b0, c0, pl.ds(0, PROWS), :], so).wait()
            pltpu.make_async_copy(
                ob, out_hbm.at[b0, c0 + 1, pl.ds(0, PROWS), :], so).wait()


_params = pltpu.CompilerParams(use_tc_tiling_on_sc=False,
                               needs_layout_passes=False)

_cidx = functools.partial(
    pl.kernel,
    out_type=jax.ShapeDtypeStruct((HW,), jnp.int32),
    mesh=_mesh,
    compiler_params=_params,
    scratch_types=[
        pltpu.VMEM((V,), jnp.int32),
        pltpu.VMEM((PIX_SUB // W, W), jnp.int32),
        pltpu.VMEM((PIX_SUB,), jnp.int32),
    ],
)(_cidx_body)

_relay = functools.partial(
    pl.kernel,
    out_type=jax.ShapeDtypeStruct((ROWS * V,), jnp.float32),
    mesh=_mesh,
    compiler_params=_params,
    scratch_types=[
        pltpu.VMEM((8, VCH), jnp.float32),
        pltpu.SemaphoreType.DMA,
    ],
)(_relay_body)

_gather = functools.partial(
    pl.kernel,
    out_type=jax.ShapeDtypeStruct((B, C, H, W), jnp.float32),
    mesh=_mesh,
    compiler_params=_params,
    scratch_types=[
        pltpu.VMEM((V,), jnp.float32),
        pltpu.VMEM((V,), jnp.float32),
        pltpu.VMEM((PCH,), jnp.int32),
        pltpu.VMEM((PCH,), jnp.int32),
        pltpu.VMEM((PROWS, W), jnp.float32),
        pltpu.VMEM((PROWS, W), jnp.float32),
        pltpu.VMEM((PROWS, W), jnp.float32),
        pltpu.VMEM((PROWS, W), jnp.float32),
        pltpu.SemaphoreType.DMA,
        pltpu.SemaphoreType.DMA,
        pltpu.SemaphoreType.DMA,
        pltpu.SemaphoreType.DMA,
    ],
)(_gather_body)


@jax.jit
def kernel(vertex_values, indices, v2i_idx):
    ind32 = indices if indices.dtype == jnp.int32 else indices.astype(jnp.int32)
    v2i = v2i_idx if v2i_idx.dtype == jnp.int32 else v2i_idx.astype(jnp.int32)
    cidx = _cidx(ind32, v2i)
    vvt = jnp.transpose(vertex_values, (0, 2, 1))
    tblt = _relay(vvt, lax.slice(vvt, (0, 0, VT0), (B, C, V)))
    return _gather(tblt, cidx)
